# baseline (speedup 1.0000x reference)
import functools

import jax
import jax.numpy as jnp
from jax import lax
from jax.experimental import pallas as pl
from jax.experimental.pallas import tpu as pltpu
from jax.experimental.pallas import tpu_sc as plsc

GAMMA = 0.99

B = 16384
A = 1024
RB = 1024
G = 4
GROWS = 3072
SC_ROWS = G * GROWS
TAIL = B - SC_ROWS
NBLKG = GROWS // RB

NC = 2
NS = 16
NW = NC * NS
PER_W = GROWS // NW
L = 16
CROWS = 16
NCHUNK = PER_W // CROWS
NBUF = 4


def _argmax_body(av_ref, out_ref):
    av = av_ref[...]
    mx = jnp.max(av, axis=1, keepdims=True)
    cols = lax.broadcasted_iota(jnp.int32, (RB, A), 1)
    big = jnp.int32(2**30)
    cand = jnp.where(av == mx, cols, big)
    out_ref[0, 0, :] = jnp.min(cand, axis=1)


def _argmax_cols(action_values, g):
    out = pl.pallas_call(
        _argmax_body,
        grid=(NBLKG,),
        in_specs=[pl.BlockSpec((RB, A), lambda i, g=g: (g * NBLKG + i, 0))],
        out_specs=pl.BlockSpec((1, 1, RB), lambda i: (i, 0, 0)),
        out_shape=jax.ShapeDtypeStruct((NBLKG, 1, RB), jnp.int32),
    )(action_values)
    return out.reshape(GROWS)


def _sc_body(g, actual_hbm, cidx_hbm, rew_hbm, done_hbm, out_hbm,
             cidx_v, buf0_v, buf1_v, buf2_v, buf3_v, rew_v, done_v, out_v,
             sem0, sem1, sem2, sem3):
    wid = lax.axis_index("s") * NC + lax.axis_index("c")
    lbase = wid * PER_W
    base = g * GROWS + lbase
    pltpu.sync_copy(cidx_hbm.at[pl.ds(lbase, PER_W)], cidx_v)
    pltpu.sync_copy(rew_hbm.at[pl.ds(base, PER_W)], rew_v)
    pltpu.sync_copy(done_hbm.at[pl.ds(base, PER_W)], done_v)
    lanes = lax.iota(jnp.int32, L)

    bufs = [buf0_v, buf1_v, buf2_v, buf3_v]
    sems = [sem0, sem1, sem2, sem3]
    descs = [None] * NCHUNK
    for k in range(NBUF - 1):
        descs[k] = pltpu.async_copy(
            actual_hbm.at[pl.ds(base + k * CROWS, CROWS)],
            bufs[k % NBUF], sems[k % NBUF])
    for k in range(NCHUNK):
        if k + NBUF - 1 < NCHUNK:
            kk = k + NBUF - 1
            descs[kk] = pltpu.async_copy(
                actual_hbm.at[pl.ds(base + kk * CROWS, CROWS)],
                bufs[kk % NBUF], sems[kk % NBUF])
        descs[k].wait()
        buf = bufs[k % NBUF]
        for h in range(CROWS // L):
            sl = pl.ds(k * CROWS + h * L, L)
            cvec = cidx_v[sl]
            lr = lanes + h * L
            v = plsc.load_gather(buf, [lr, cvec])
            dn = done_v[sl]
            rw = rew_v[sl]
            w = jnp.where(dn != jnp.float32(0.0), jnp.float32(0.0), v)
            out_v[sl] = w * jnp.float32(GAMMA) + rw
    pltpu.sync_copy(out_v, out_hbm.at[pl.ds(lbase, PER_W)])


def _sc_select_epilogue(actual, cidx_g, rew, done_f, g):
    mesh = plsc.VectorSubcoreMesh(
        core_axis_name="c", subcore_axis_name="s",
        num_cores=NC, num_subcores=NS,
    )
    f = functools.partial(
        pl.kernel,
        mesh=mesh,
        out_type=jax.ShapeDtypeStruct((GROWS,), jnp.float32),
        scratch_types=[
            pltpu.VMEM((PER_W,), jnp.int32),
            pltpu.VMEM((CROWS, A), jnp.float32),
            pltpu.VMEM((CROWS, A), jnp.float32),
            pltpu.VMEM((CROWS, A), jnp.float32),
            pltpu.VMEM((CROWS, A), jnp.float32),
            pltpu.VMEM((PER_W,), jnp.float32),
            pltpu.VMEM((PER_W,), jnp.float32),
            pltpu.VMEM((PER_W,), jnp.float32),
            pltpu.SemaphoreType.DMA,
            pltpu.SemaphoreType.DMA,
            pltpu.SemaphoreType.DMA,
            pltpu.SemaphoreType.DMA,
        ],
        compiler_params=pltpu.CompilerParams(
            use_tc_tiling_on_sc=True, needs_layout_passes=False, skip_device_barrier=True, has_side_effects=False),
    )(functools.partial(_sc_body, g))
    return f(actual, cidx_g, rew, done_f)


def _fused_body(actual_ref, action_ref, rew_ref, done_ref, out_ref):
    av = action_ref[...]
    ac = actual_ref[...]
    mx = jnp.max(av, axis=1, keepdims=True)
    cols = lax.broadcasted_iota(jnp.int32, (RB, A), 1)
    big = jnp.int32(2**30)
    cand = jnp.where(av == mx, cols, big)
    cstar = jnp.min(cand, axis=1, keepdims=True)
    val = jnp.sum(jnp.where(cand == cstar, ac, jnp.float32(0.0)),
                  axis=1, keepdims=True)
    dn = done_ref[...]
    rw = rew_ref[...]
    w = jnp.where(dn != jnp.float32(0.0), jnp.float32(0.0), val)
    out_ref[...] = w * jnp.float32(GAMMA) + rw


def _fused_tail(actual, action, reward2d, done2d):
    off = SC_ROWS // RB
    out = pl.pallas_call(
        _fused_body,
        grid=(TAIL // RB,),
        in_specs=[
            pl.BlockSpec((RB, A), lambda i: (off + i, 0)),
            pl.BlockSpec((RB, A), lambda i: (off + i, 0)),
            pl.BlockSpec((RB, 1), lambda i: (off + i, 0)),
            pl.BlockSpec((RB, 1), lambda i: (off + i, 0)),
        ],
        out_specs=pl.BlockSpec((RB, 1), lambda i: (i, 0)),
        out_shape=jax.ShapeDtypeStruct((TAIL, 1), jnp.float32),
    )(actual, action, reward2d, done2d)
    return out.reshape(TAIL)


def kernel(next_state_actual_values, next_state_action_values, reward, is_done):
    done2d = is_done.astype(jnp.float32)
    rew = reward.reshape(B)
    done_f = done2d.reshape(B)
    outs = []
    for g in range(G):
        cidx_g = _argmax_cols(next_state_action_values, g)
        outs.append(_sc_select_epilogue(
            next_state_actual_values, cidx_g, rew, done_f, g))
    outs.append(_fused_tail(
        next_state_actual_values, next_state_action_values, reward, done2d))
    return jnp.concatenate(outs)


# speedup vs baseline: 1.0420x; 1.0420x over previous
import functools

import jax
import jax.numpy as jnp
from jax import lax
from jax.experimental import pallas as pl
from jax.experimental.pallas import tpu as pltpu
from jax.experimental.pallas import tpu_sc as plsc

GAMMA = 0.99

B = 16384
A = 1024
RB = 1024
G = 3
GROWS = 4096
SC_ROWS = G * GROWS
TAIL = B - SC_ROWS
NBLKG = GROWS // RB

NC = 2
NS = 16
NW = NC * NS
PER_W = GROWS // NW
WPB = RB // PER_W
L = 16
CROWS = 16
NCHUNK = PER_W // CROWS
NBUF = 4


def _argmax_body(av_ref, rew_ref, done_ref, cidx_ref, scale_ref, bias_ref):
    av = av_ref[...]
    mx = jnp.max(av, axis=1, keepdims=True)
    cols = lax.broadcasted_iota(jnp.int32, (RB, A), 1)
    big = jnp.int32(2**30)
    cand = jnp.where(av == mx, cols, big)
    cidx_ref[0, 0, :] = jnp.min(cand, axis=1)
    dn = done_ref[...]
    rw = rew_ref[...]
    sc = jnp.where(dn != jnp.float32(0.0), jnp.float32(0.0), jnp.float32(GAMMA))
    scale_ref[0, 0, :] = sc[:, 0]
    bias_ref[0, 0, :] = rw[:, 0]


def _argmax_cols(action_values, rew2d, done2d, g):
    shp = jax.ShapeDtypeStruct((NBLKG, 1, RB), jnp.float32)
    return pl.pallas_call(
        _argmax_body,
        grid=(NBLKG,),
        in_specs=[
            pl.BlockSpec((RB, A), lambda i, g=g: (g * NBLKG + i, 0)),
            pl.BlockSpec((RB, 1), lambda i, g=g: (g * NBLKG + i, 0)),
            pl.BlockSpec((RB, 1), lambda i, g=g: (g * NBLKG + i, 0)),
        ],
        out_specs=[
            pl.BlockSpec((1, 1, RB), lambda i: (i, 0, 0)),
            pl.BlockSpec((1, 1, RB), lambda i: (i, 0, 0)),
            pl.BlockSpec((1, 1, RB), lambda i: (i, 0, 0)),
        ],
        out_shape=[
            jax.ShapeDtypeStruct((NBLKG, 1, RB), jnp.int32), shp, shp,
        ],
    )(action_values, rew2d, done2d)


def _sc_body(g, actual_hbm, cidx_hbm, scale_hbm, bias_hbm, out_hbm,
             cidx_v, buf0_v, buf1_v, buf2_v, buf3_v, scale_v, bias_v, out_v,
             sem0, sem1, sem2, sem3):
    wid = lax.axis_index("s") * NC + lax.axis_index("c")
    lbase = wid * PER_W
    base = g * GROWS + lbase
    blk = wid // WPB
    off = (wid % WPB) * PER_W
    pltpu.sync_copy(cidx_hbm.at[blk, 0, pl.ds(off, PER_W)], cidx_v)
    pltpu.sync_copy(scale_hbm.at[blk, 0, pl.ds(off, PER_W)], scale_v)
    pltpu.sync_copy(bias_hbm.at[blk, 0, pl.ds(off, PER_W)], bias_v)
    lanes = lax.iota(jnp.int32, L)

    bufs = [buf0_v, buf1_v, buf2_v, buf3_v]
    sems = [sem0, sem1, sem2, sem3]
    descs = [None] * NCHUNK
    for k in range(NBUF - 1):
        descs[k] = pltpu.async_copy(
            actual_hbm.at[pl.ds(base + k * CROWS, CROWS)],
            bufs[k % NBUF], sems[k % NBUF])
    for k in range(NCHUNK):
        if k + NBUF - 1 < NCHUNK:
            kk = k + NBUF - 1
            descs[kk] = pltpu.async_copy(
                actual_hbm.at[pl.ds(base + kk * CROWS, CROWS)],
                bufs[kk % NBUF], sems[kk % NBUF])
        descs[k].wait()
        buf = bufs[k % NBUF]
        for h in range(CROWS // L):
            sl = pl.ds(k * CROWS + h * L, L)
            cvec = cidx_v[sl]
            lr = lanes + h * L
            v = plsc.load_gather(buf, [lr, cvec])
            out_v[sl] = v * scale_v[sl] + bias_v[sl]
    pltpu.sync_copy(out_v, out_hbm.at[pl.ds(lbase, PER_W)])


def _sc_select_epilogue(actual, cidx3, scale3, bias3, g):
    mesh = plsc.VectorSubcoreMesh(
        core_axis_name="c", subcore_axis_name="s",
        num_cores=NC, num_subcores=NS,
    )
    f = functools.partial(
        pl.kernel,
        mesh=mesh,
        out_type=jax.ShapeDtypeStruct((GROWS,), jnp.float32),
        scratch_types=[
            pltpu.VMEM((PER_W,), jnp.int32),
            pltpu.VMEM((CROWS, A), jnp.float32),
            pltpu.VMEM((CROWS, A), jnp.float32),
            pltpu.VMEM((CROWS, A), jnp.float32),
            pltpu.VMEM((CROWS, A), jnp.float32),
            pltpu.VMEM((PER_W,), jnp.float32),
            pltpu.VMEM((PER_W,), jnp.float32),
            pltpu.VMEM((PER_W,), jnp.float32),
            pltpu.SemaphoreType.DMA,
            pltpu.SemaphoreType.DMA,
            pltpu.SemaphoreType.DMA,
            pltpu.SemaphoreType.DMA,
        ],
        compiler_params=pltpu.CompilerParams(
            use_tc_tiling_on_sc=True, needs_layout_passes=False,
            skip_device_barrier=True, has_side_effects=False),
    )(functools.partial(_sc_body, g))
    return f(actual, cidx3, scale3, bias3)


def _fused_body(actual_ref, action_ref, rew_ref, done_ref, out_ref):
    av = action_ref[...]
    ac = actual_ref[...]
    mx = jnp.max(av, axis=1, keepdims=True)
    cols = lax.broadcasted_iota(jnp.int32, (RB, A), 1)
    big = jnp.int32(2**30)
    cand = jnp.where(av == mx, cols, big)
    cstar = jnp.min(cand, axis=1, keepdims=True)
    val = jnp.sum(jnp.where(cand == cstar, ac, jnp.float32(0.0)),
                  axis=1, keepdims=True)
    dn = done_ref[...]
    rw = rew_ref[...]
    w = jnp.where(dn != jnp.float32(0.0), jnp.float32(0.0), val)
    out_ref[...] = w * jnp.float32(GAMMA) + rw


def _fused_tail(actual, action, rew2d, done2d):
    off = SC_ROWS // RB
    out = pl.pallas_call(
        _fused_body,
        grid=(TAIL // RB,),
        in_specs=[
            pl.BlockSpec((RB, A), lambda i: (off + i, 0)),
            pl.BlockSpec((RB, A), lambda i: (off + i, 0)),
            pl.BlockSpec((RB, 1), lambda i: (off + i, 0)),
            pl.BlockSpec((RB, 1), lambda i: (off + i, 0)),
        ],
        out_specs=pl.BlockSpec((RB, 1), lambda i: (i, 0)),
        out_shape=jax.ShapeDtypeStruct((TAIL, 1), jnp.float32),
    )(actual, action, rew2d, done2d)
    return out.reshape(TAIL)


def kernel(next_state_actual_values, next_state_action_values, reward, is_done):
    done2d = is_done.astype(jnp.float32)
    outs = []
    for g in range(G):
        cidx3, scale3, bias3 = _argmax_cols(
            next_state_action_values, reward, done2d, g)
        outs.append(_sc_select_epilogue(
            next_state_actual_values, cidx3, scale3, bias3, g))
    outs.append(_fused_tail(
        next_state_actual_values, next_state_action_values, reward, done2d))
    return jnp.concatenate(outs)


# speedup vs baseline: 2.0450x; 1.9626x over previous
import jax
import jax.numpy as jnp
from jax import lax
from jax.experimental import pallas as pl

GAMMA = 0.99

B = 16384
A = 1024
RB = 1024
NBLK = B // RB


def _body(actual_ref, action_ref, rew_ref, done_ref, out_ref):
    av = action_ref[...]
    ac = actual_ref[...]
    mx = jnp.max(av, axis=1, keepdims=True)
    cols = lax.broadcasted_iota(jnp.int32, (RB, A), 1)
    big = jnp.int32(2**30)
    cand = jnp.where(av == mx, cols, big)
    cstar = jnp.min(cand, axis=1, keepdims=True)
    val = jnp.sum(jnp.where(cand == cstar, ac, jnp.float32(0.0)),
                  axis=1, keepdims=True)
    vl = val.reshape(1, 1, RB)
    dl = done_ref[...].reshape(1, 1, RB)
    rl = rew_ref[...].reshape(1, 1, RB)
    w = jnp.where(dl != jnp.float32(0.0), jnp.float32(0.0), vl)
    out_ref[...] = w * jnp.float32(GAMMA) + rl


def kernel(next_state_actual_values, next_state_action_values, reward, is_done):
    rew1 = reward.reshape(B)
    done1 = is_done.astype(jnp.float32).reshape(B)
    out = pl.pallas_call(
        _body,
        grid=(NBLK,),
        in_specs=[
            pl.BlockSpec((RB, A), lambda i: (i, 0)),
            pl.BlockSpec((RB, A), lambda i: (i, 0)),
            pl.BlockSpec((RB,), lambda i: (i,)),
            pl.BlockSpec((RB,), lambda i: (i,)),
        ],
        out_specs=pl.BlockSpec((1, 1, RB), lambda i: (i, 0, 0)),
        out_shape=jax.ShapeDtypeStruct((NBLK, 1, RB), jnp.float32),
    )(next_state_actual_values, next_state_action_values, rew1, done1)
    return out.reshape(B)
